# baseline (speedup 1.0000x reference)
import functools

import jax
import jax.numpy as jnp
from jax.experimental import pallas as pl
from jax.experimental.pallas import tpu as pltpu

_B, _S, _D, _LD = 4, 8192, 1024, 48
_TOK_BLK = 2048
_SPB = _S // _TOK_BLK


def _gelu_exact(v):
    return 0.5 * v * (1.0 + jax.lax.erf(v * 0.7071067811865476))


def _block_body(xh_ref, x_ref, lat_ref, wnt_ref, bnt_ref, lnw_ref, lnb_ref,
                we1_ref, be1_ref, we2_ref, be2_ref, wout_ref, bout_ref,
                wg_ref, bg_ref, out_ref, uph):
    out_ref[...] = x_ref[...]

    t = pl.program_id(0)

    def _compute(b):
        r0 = pl.multiple_of(b * _LD, 8)
        xl = xh_ref[pl.ds(r0, _LD), :]
        h = jnp.dot(xl, wnt_ref[...], preferred_element_type=jnp.float32)
        h = h + bnt_ref[...]
        mu = jnp.mean(h, axis=-1, keepdims=True)
        var = jnp.mean((h - mu) ** 2, axis=-1, keepdims=True)
        h = (h - mu) / jnp.sqrt(var + 1e-5) * lnw_ref[...] + lnb_ref[...]
        h = _gelu_exact(h)

        lat = lat_ref[...]
        lat = lat - jnp.max(lat, axis=-1, keepdims=True)
        e = jnp.exp(lat)
        adj = e / jnp.sum(e, axis=-1, keepdims=True)
        w_masked = jnp.where(adj > 0.01, adj, 0.0)
        wn = jnp.dot(w_masked, h, preferred_element_type=jnp.float32)

        msg = (jnp.dot(h, we1_ref[:_D, :], preferred_element_type=jnp.float32)
               + jnp.dot(wn, we1_ref[_D:, :], preferred_element_type=jnp.float32)
               + be1_ref[...])
        msg = _gelu_exact(msg)
        msg = jnp.dot(msg, we2_ref[...], preferred_element_type=jnp.float32) + be2_ref[...]

        g = jax.nn.sigmoid(
            jnp.dot(xl, wg_ref[:_D, :], preferred_element_type=jnp.float32)
            + jnp.dot(msg, wg_ref[_D:, :], preferred_element_type=jnp.float32)
            + bg_ref[...])
        upd = g * (jnp.dot(msg, wout_ref[...], preferred_element_type=jnp.float32)
                   + bout_ref[...]) + (1.0 - g) * xl
        uph[pl.ds(r0, _LD), :] = upd

    @pl.when(t == 0)
    def _():
        _compute(0)
        out_ref[:_LD, :] = uph[:_LD, :]

    @pl.when(jnp.logical_and(t % _SPB == _SPB - 2, t < (_B - 1) * _SPB))
    def _():
        _compute(t // _SPB + 1)

    @pl.when(jnp.logical_and(t % _SPB == 0, t > 0))
    def _():
        b = t // _SPB
        r0 = pl.multiple_of(b * _LD, 8)
        out_ref[:_LD, :] = uph[pl.ds(r0, _LD), :]


@functools.partial(jax.jit, static_argnames=("interpret",))
def _run(x, lattice_weights, W_nt, b_nt, ln_w, ln_b, W_e1, b_e1, W_e2, b_e2,
         W_out, b_out, W_g, b_g, interpret=False):
    B, S, D = x.shape
    xf = x.reshape(B * S, D)
    xh = x[:, :_LD, :].reshape(B * _LD, D)
    grid = (B * S // _TOK_BLK,)
    full = lambda a: pl.BlockSpec(a.shape, lambda t: (0,) * a.ndim)
    out = pl.pallas_call(
        _block_body,
        grid=grid,
        in_specs=[
            full(xh),
            pl.BlockSpec((_TOK_BLK, D), lambda t: (t, 0)),
            full(lattice_weights), full(W_nt), full(b_nt), full(ln_w),
            full(ln_b), full(W_e1), full(b_e1), full(W_e2), full(b_e2),
            full(W_out), full(b_out), full(W_g), full(b_g),
        ],
        out_specs=pl.BlockSpec((_TOK_BLK, D), lambda t: (t, 0)),
        out_shape=jax.ShapeDtypeStruct((B * S, D), x.dtype),
        scratch_shapes=[pltpu.VMEM((B * _LD, D), jnp.float32)],
        compiler_params=pltpu.CompilerParams(
            dimension_semantics=("arbitrary",),
            vmem_limit_bytes=100 * 1024 * 1024),
        interpret=interpret,
    )(xh, xf, lattice_weights, W_nt, b_nt, ln_w, ln_b, W_e1, b_e1, W_e2,
      b_e2, W_out, b_out, W_g, b_g)
    return out.reshape(B, S, D)


def kernel(x, lattice_weights, W_nt, b_nt, ln_w, ln_b, W_e1, b_e1, W_e2,
           b_e2, W_out, b_out, W_g, b_g):
    return _run(x, lattice_weights, W_nt, b_nt, ln_w, ln_b, W_e1, b_e1,
                W_e2, b_e2, W_out, b_out, W_g, b_g)


# speedup vs baseline: 1.0219x; 1.0219x over previous
import functools

import jax
import jax.numpy as jnp
from jax.experimental import pallas as pl
from jax.experimental.pallas import tpu as pltpu

_B, _S, _D, _LD = 4, 8192, 1024, 48
_TOK_BLK = 2048


def _gelu_exact(v):
    return 0.5 * v * (1.0 + jax.lax.erf(v * 0.7071067811865476))


def _block_body(x_ref, lat_ref, wnt_ref, bnt_ref, lnw_ref, lnb_ref,
                we1_ref, be1_ref, we2_ref, be2_ref, wout_ref, bout_ref,
                wg_ref, bg_ref, out_ref):
    out_ref[...] = x_ref[...]

    @pl.when(pl.program_id(0) % (_S // _TOK_BLK) == 0)
    def _compute():
        L = _LD
        xl = x_ref[:L, :]
        h = jnp.dot(xl, wnt_ref[...], preferred_element_type=jnp.float32)
        h = h + bnt_ref[...]
        mu = jnp.mean(h, axis=-1, keepdims=True)
        var = jnp.mean((h - mu) ** 2, axis=-1, keepdims=True)
        h = (h - mu) / jnp.sqrt(var + 1e-5) * lnw_ref[...] + lnb_ref[...]
        h = _gelu_exact(h)

        lat = lat_ref[...]
        lat = lat - jnp.max(lat, axis=-1, keepdims=True)
        e = jnp.exp(lat)
        adj = e / jnp.sum(e, axis=-1, keepdims=True)
        w_masked = jnp.where(adj > 0.01, adj, 0.0)
        wn = jnp.dot(w_masked, h, preferred_element_type=jnp.float32)

        msg = (jnp.dot(h, we1_ref[:_D, :], preferred_element_type=jnp.float32)
               + jnp.dot(wn, we1_ref[_D:, :], preferred_element_type=jnp.float32)
               + be1_ref[...])
        msg = _gelu_exact(msg)
        msg = jnp.dot(msg, we2_ref[...], preferred_element_type=jnp.float32) + be2_ref[...]

        g = jax.nn.sigmoid(
            jnp.dot(xl, wg_ref[:_D, :], preferred_element_type=jnp.float32)
            + jnp.dot(msg, wg_ref[_D:, :], preferred_element_type=jnp.float32)
            + bg_ref[...])
        upd = g * (jnp.dot(msg, wout_ref[...], preferred_element_type=jnp.float32)
                   + bout_ref[...]) + (1.0 - g) * xl
        out_ref[:L, :] = upd


@functools.partial(jax.jit, static_argnames=("interpret",))
def _run(x, lattice_weights, W_nt, b_nt, ln_w, ln_b, W_e1, b_e1, W_e2, b_e2,
         W_out, b_out, W_g, b_g, interpret=False):
    B, S, D = x.shape
    xf = x.reshape(B * S, D)
    grid = (B * S // _TOK_BLK,)
    full = lambda a: pl.BlockSpec(a.shape, lambda t: (0,) * a.ndim)
    out = pl.pallas_call(
        _block_body,
        grid=grid,
        in_specs=[
            pl.BlockSpec((_TOK_BLK, D), lambda t: (t, 0)),
            full(lattice_weights), full(W_nt), full(b_nt), full(ln_w),
            full(ln_b), full(W_e1), full(b_e1), full(W_e2), full(b_e2),
            full(W_out), full(b_out), full(W_g), full(b_g),
        ],
        out_specs=pl.BlockSpec((_TOK_BLK, D), lambda t: (t, 0)),
        out_shape=jax.ShapeDtypeStruct((B * S, D), x.dtype),
        compiler_params=pltpu.CompilerParams(
            dimension_semantics=("parallel",),
            vmem_limit_bytes=100 * 1024 * 1024),
        interpret=interpret,
    )(xf, lattice_weights, W_nt, b_nt, ln_w, ln_b, W_e1, b_e1, W_e2, b_e2,
      W_out, b_out, W_g, b_g)
    return out.reshape(B, S, D)


def kernel(x, lattice_weights, W_nt, b_nt, ln_w, ln_b, W_e1, b_e1, W_e2,
           b_e2, W_out, b_out, W_g, b_g):
    return _run(x, lattice_weights, W_nt, b_nt, ln_w, ln_b, W_e1, b_e1,
                W_e2, b_e2, W_out, b_out, W_g, b_g)
